# Initial kernel scaffold; baseline (speedup 1.0000x reference)
import jax
import jax.numpy as jnp
from jax.experimental import pallas as pl


def kernel(x, x_proj_weight, dt_projs_weight, dt_projs_bias, A_logs, Ds, rw1, rb1, rw2, rb2):
    raise NotImplementedError("write your pallas kernel here")


# speedup vs baseline: 5.9353x; 5.9353x over previous
import functools
import math

import jax
import jax.numpy as jnp
from jax.experimental import pallas as pl
from jax.experimental.pallas import tpu as pltpu

B_, C_, H_, W_ = 2, 384, 224, 224
WIN = 7
K_RATIO = 0.25
D_STATE = 16
DT_RANK = 24
KDIR = 4

NH = H_ // WIN
NW = W_ // WIN
N_WINDOWS = NH * NW
TOP_K = 256
GRID_N = 16
L_SCAN = TOP_K * WIN * WIN
DLANE = B_ * C_

SCAN_T = 64


def _scan_chunk_kernel(u_ref, draw_ref, b_ref, c_ref, aarr_ref, dtb_ref,
                       ds_ref, e2_ref, y_ref, h_ref, hbuf_ref, da_ref, dbu_ref):
    T = y_ref.shape[0]

    @pl.when(pl.program_id(0) == 0)
    def _():
        h_ref[...] = jnp.zeros_like(h_ref)

    delta = jax.nn.softplus(draw_ref[...] + dtb_ref[0][None, :])
    du = delta * u_ref[...]
    dA = jnp.exp(delta[:, None, :] * aarr_ref[...][None, :, :])
    dn = (((2,), (0,)), ((), ()))
    Bex = jax.lax.dot_general(b_ref[...], e2_ref[...], dn,
                              preferred_element_type=jnp.float32)
    Cex = jax.lax.dot_general(c_ref[...], e2_ref[...], dn,
                              preferred_element_type=jnp.float32)
    da_ref[...] = dA
    dbu_ref[...] = du[:, None, :] * Bex

    def body(t, h):
        h = da_ref[t] * h + dbu_ref[t]
        hbuf_ref[t] = h
        return h

    h = jax.lax.fori_loop(0, T, body, h_ref[...])
    h_ref[...] = h
    y_ref[...] = jnp.sum(hbuf_ref[...] * Cex, axis=1) + u_ref[...] * ds_ref[0][None, :]


def _selective_scan_pallas(u2, draw2, b2, c2, aarr, dtb2, ds2, e2):
    L = u2.shape[0]
    T = SCAN_T
    grid = (L // T,)
    return pl.pallas_call(
        _scan_chunk_kernel,
        grid=grid,
        in_specs=[
            pl.BlockSpec((T, DLANE), lambda i: (i, 0)),
            pl.BlockSpec((T, DLANE), lambda i: (i, 0)),
            pl.BlockSpec((T, 16, 8), lambda i: (i, 0, 0)),
            pl.BlockSpec((T, 16, 8), lambda i: (i, 0, 0)),
            pl.BlockSpec((16, DLANE), lambda i: (0, 0)),
            pl.BlockSpec((1, DLANE), lambda i: (0, 0)),
            pl.BlockSpec((1, DLANE), lambda i: (0, 0)),
            pl.BlockSpec((8, DLANE), lambda i: (0, 0)),
        ],
        out_specs=pl.BlockSpec((T, DLANE), lambda i: (i, 0)),
        out_shape=jax.ShapeDtypeStruct((L, DLANE), jnp.float32),
        scratch_shapes=[
            pltpu.VMEM((16, DLANE), jnp.float32),
            pltpu.VMEM((T, 16, DLANE), jnp.float32),
            pltpu.VMEM((T, 16, DLANE), jnp.float32),
            pltpu.VMEM((T, 16, DLANE), jnp.float32),
        ],
    )(u2, draw2, b2, c2, aarr, dtb2, ds2, e2)


def _local_scan(t, Bsz, c, flip=False, column_first=False):
    if column_first:
        t = jnp.transpose(t, (0, 3, 2, 1, 5, 4)).reshape(Bsz, c, -1)
    else:
        t = jnp.transpose(t, (0, 3, 1, 2, 4, 5)).reshape(Bsz, c, -1)
    if flip:
        t = t[..., ::-1]
    return t


def _local_reverse(t, nH, nW, wH, wW, flip=False, column_first=False):
    Bsz, c, L = t.shape
    if flip:
        t = t[..., ::-1]
    if column_first:
        t = jnp.transpose(t.reshape(Bsz, c, nW, nH, wW, wH), (0, 1, 3, 5, 2, 4)).reshape(Bsz, c, L)
    else:
        t = jnp.transpose(t.reshape(Bsz, c, nH, nW, wH, wW), (0, 1, 2, 4, 3, 5)).reshape(Bsz, c, L)
    return t


def kernel(x, x_proj_weight, dt_projs_weight, dt_projs_bias, A_logs, Ds,
           rw1, rb1, rw2, rb2):
    B, C, H, W = x.shape
    n = GRID_N
    L = L_SCAN

    windows = jnp.transpose(
        x.reshape(B, C, NH, WIN, NW, WIN), (0, 2, 4, 1, 3, 5)
    ).reshape(B, N_WINDOWS, C, WIN, WIN)
    pooled = windows.mean(axis=(-2, -1))
    h = jax.nn.gelu(pooled @ rw1.T + rb1, approximate=False)
    router_logits = (h @ rw2.T + rb2)[..., 0]
    orig_rw = jax.nn.softmax(router_logits, axis=1)
    routing_weights, sel = jax.lax.top_k(orig_rw, TOP_K)

    windows_flat = windows.reshape(B, N_WINDOWS, -1)
    current = jnp.take_along_axis(windows_flat, sel[:, :, None], axis=1)
    current = current.reshape(B, n, n, C, WIN, WIN)
    cs = jnp.transpose(
        current.reshape(B, n, n, C // 4, 4, WIN, WIN), (0, 1, 2, 4, 3, 5, 6)
    ).reshape(B, n, n, C, WIN, WIN)
    x_split = jnp.split(cs, 4, axis=3)
    xs = [
        _local_scan(x_split[0], B, C // 4, flip=False, column_first=False),
        _local_scan(x_split[1], B, C // 4, flip=False, column_first=True),
        _local_scan(x_split[2], B, C // 4, flip=True, column_first=False),
        _local_scan(x_split[3], B, C // 4, flip=True, column_first=True),
    ]
    xs = jnp.stack(xs, axis=1).reshape(B, 4, -1, L)

    x_dbl = jnp.einsum('bkdl,kcd->bkcl', xs, x_proj_weight)
    dts, Bs, Cs = jnp.split(x_dbl, [DT_RANK, DT_RANK + D_STATE], axis=2)
    dts = jnp.einsum('bkrl,kdr->bkdl', dts, dt_projs_weight)

    u2 = jnp.transpose(xs.reshape(B, C, L), (2, 0, 1)).reshape(L, B * C)
    draw2 = jnp.transpose(dts.reshape(B, C, L), (2, 0, 1)).reshape(L, B * C)
    b2 = jnp.transpose(Bs, (3, 2, 0, 1)).reshape(L, 16, B * 4)
    c2 = jnp.transpose(Cs, (3, 2, 0, 1)).reshape(L, 16, B * 4)
    A = -jnp.exp(A_logs)
    aarr = jnp.tile(A.T, (1, B))
    dtb2 = jnp.tile(dt_projs_bias.reshape(1, C), (1, B))
    ds2 = jnp.tile(Ds.reshape(1, C), (1, B))
    lane = jnp.arange(B * C) // (C // 4)
    e2 = (jnp.arange(B * 4)[:, None] == lane[None, :]).astype(jnp.float32)

    y2 = _selective_scan_pallas(u2, draw2, b2, c2, aarr, dtb2, ds2, e2)
    out_y = jnp.transpose(y2.reshape(L, B, C), (1, 2, 0)).reshape(B, 4, C // 4, L)

    ys = [
        _local_reverse(out_y[:, 0], n, n, WIN, WIN, flip=False, column_first=False),
        _local_reverse(out_y[:, 1], n, n, WIN, WIN, flip=False, column_first=True),
        _local_reverse(out_y[:, 2], n, n, WIN, WIN, flip=True, column_first=False),
        _local_reverse(out_y[:, 3], n, n, WIN, WIN, flip=True, column_first=True),
    ]
    y = jnp.concatenate(ys, axis=1)
    y = jnp.transpose(y.reshape(B, C, n * n, WIN * WIN), (0, 2, 1, 3)).reshape(B, TOP_K, -1)
    current_state = y * routing_weights[:, :, None]
    residual_x = windows_flat * orig_rw[:, :, None]
    residual_x = residual_x.at[jnp.arange(B)[:, None], sel].set(current_state)
    out = jnp.transpose(
        residual_x.reshape(B, NH, NW, C, WIN, WIN), (0, 3, 1, 4, 2, 5)
    ).reshape(B, C, H, W)
    return out


# speedup vs baseline: 6.5041x; 1.0958x over previous
import functools
import math

import jax
import jax.numpy as jnp
import numpy as np
from jax.experimental import pallas as pl
from jax.experimental.pallas import tpu as pltpu

B_, C_, H_, W_ = 2, 384, 224, 224
WIN = 7
D_STATE = 16
DT_RANK = 24
KDIR = 4
PERK = C_ // KDIR

NH = H_ // WIN
NW = W_ // WIN
N_WINDOWS = NH * NW
TOP_K = 256
GRID_N = 16
L_SCAN = TOP_K * WIN * WIN
NG = 2 * KDIR
GL = 128
DLANE = NG * GL

SCAN_T = 64
GWIN = 8


def _windowize_kernel(x_ref, i96_ref, xw_ref, pool_ref):
    xin = x_ref[0, :, 0]
    dn = (((0,), (0,)), ((), ()))
    t = jax.lax.dot_general(xin, i96_ref[...], dn,
                            preferred_element_type=jnp.float32)
    w = t.reshape(8, 7, 32, 7, GL).transpose(0, 2, 1, 3, 4).reshape(256, 49, GL)
    xw_ref[0, 0] = w
    pool_ref[0, 0, 0] = jnp.sum(w, axis=1) * np.float32(1.0 / 49.0)


def _windowize(x5, i96):
    return pl.pallas_call(
        _windowize_kernel,
        grid=(B_, KDIR, 4),
        in_specs=[
            pl.BlockSpec((1, PERK, 1, 56, W_), lambda b, k, hb: (b, 0, k, hb, 0)),
            pl.BlockSpec((PERK, GL), lambda b, k, hb: (0, 0)),
        ],
        out_specs=[
            pl.BlockSpec((1, 1, 256, 49, GL), lambda b, k, hb: (b, k, hb, 0, 0)),
            pl.BlockSpec((1, 1, 1, 256, GL), lambda b, k, hb: (b, k, hb, 0, 0)),
        ],
        out_shape=[
            jax.ShapeDtypeStruct((B_, KDIR, N_WINDOWS, 49, GL), jnp.float32),
            jax.ShapeDtypeStruct((B_, KDIR, 4, 256, GL), jnp.float32),
        ],
    )(x5, i96)


def _route_kernel(pool_ref, rw1_ref, rb1_ref, rw2_ref, rb2_ref,
                  rw_ref, self_ref, wsel_ref):
    dnt = (((1,), (1,)), ((), ()))
    pw = pool_ref[0].reshape(KDIR, N_WINDOWS, GL)
    h = jnp.zeros((PERK, N_WINDOWS), jnp.float32)
    for k in range(KDIR):
        h = h + jax.lax.dot_general(rw1_ref[:, k, :], pw[k], dnt,
                                    preferred_element_type=jnp.float32)
    h = h + rb1_ref[:, 0][:, None]
    h = h * 0.5 * (1.0 + jax.lax.erf(h * np.float32(1.0 / math.sqrt(2.0))))
    dn = (((1,), (0,)), ((), ()))
    lg = jax.lax.dot_general(rw2_ref[...], h, dn,
                             preferred_element_type=jnp.float32)
    lg = lg + rb2_ref[0, 0]
    m = jnp.max(lg)
    e = jnp.exp(lg - m)
    rw = e / jnp.sum(e)
    rw_ref[0] = rw

    ia = jax.lax.broadcasted_iota(jnp.int32, (1024, 1024), 0)
    ib = jax.lax.broadcasted_iota(jnp.int32, (1024, 1024), 1)
    eye = (ia == ib).astype(jnp.float32)
    vcol = jax.lax.dot_general(eye, lg, dnt,
                               preferred_element_type=jnp.float32)
    gt = (vcol > lg).astype(jnp.float32)
    eq = (vcol == lg)
    mm = gt + jnp.where(eq & (ia < ib), 1.0, 0.0)
    ones = jnp.ones((1, 1024), jnp.float32)
    rank = jax.lax.dot_general(ones, mm, dn,
                               preferred_element_type=jnp.float32)
    riota = jax.lax.broadcasted_iota(jnp.int32, (TOP_K, 1024), 0).astype(jnp.float32)
    r2 = (riota == rank).astype(jnp.float32)
    icol = jax.lax.broadcasted_iota(jnp.int32, (1024, 1), 0).astype(jnp.float32)
    self_ref[0] = jax.lax.dot_general(r2, icol, dn,
                                      preferred_element_type=jnp.float32)
    rwcol = jax.lax.dot_general(eye, rw, dnt,
                                preferred_element_type=jnp.float32)
    wsel_ref[0] = jax.lax.dot_general(r2, rwcol, dn,
                                      preferred_element_type=jnp.float32)


def _route(pooled, rw1q, rb1c, rw2r, rb2r):
    return pl.pallas_call(
        _route_kernel,
        grid=(B_,),
        in_specs=[
            pl.BlockSpec((1, KDIR, 4, 256, GL), lambda b: (b, 0, 0, 0, 0)),
            pl.BlockSpec((PERK, KDIR, GL), lambda b: (0, 0, 0)),
            pl.BlockSpec((PERK, 1), lambda b: (0, 0)),
            pl.BlockSpec((1, PERK), lambda b: (0, 0)),
            pl.BlockSpec((1, 1), lambda b: (0, 0)),
        ],
        out_specs=[
            pl.BlockSpec((1, 1, 1024), lambda b: (b, 0, 0)),
            pl.BlockSpec((1, TOP_K, 1), lambda b: (b, 0, 0)),
            pl.BlockSpec((1, TOP_K, 1), lambda b: (b, 0, 0)),
        ],
        out_shape=[
            jax.ShapeDtypeStruct((B_, 1, 1024), jnp.float32),
            jax.ShapeDtypeStruct((B_, TOP_K, 1), jnp.float32),
            jax.ShapeDtypeStruct((B_, TOP_K, 1), jnp.float32),
        ],
    )(pooled, rw1q, rb1c, rw2r, rb2r)


def _gather_kernel(sel_ref, *refs):
    xw = refs[:GWIN]
    pst_ref, mk_ref = refs[GWIN], refs[GWIN + 1]
    u_ref, aux_ref = refs[GWIN + 2], refs[GWIN + 3]
    dn = (((1,), (0,)), ((), ()))
    acc = jnp.zeros((GWIN * 49, GL), jnp.float32)
    for j in range(GWIN):
        acc = acc + jax.lax.dot_general(pst_ref[0, j], xw[j][0, 0, 0], dn,
                                        preferred_element_type=jnp.float32)
    u_ref[...] = acc
    aux_ref[...] = jax.lax.dot_general(acc, mk_ref[0], dn,
                                       preferred_element_type=jnp.float32)


def _gather(sel, xwp, pstack, mks):
    def xw_map(j):
        def f(b, k, ib, sref):
            i = ib * GWIN + j
            i1 = (i % 16) * 16 + i // 16
            i2 = 255 - i
            i3 = (i2 % 16) * 16 + i2 // 16
            idx = jnp.where(k == 0, i,
                            jnp.where(k == 1, i1, jnp.where(k == 2, i2, i3)))
            return (b, k, sref[b, idx], 0, 0)
        return f

    grid_spec = pltpu.PrefetchScalarGridSpec(
        num_scalar_prefetch=1,
        grid=(B_, KDIR, TOP_K // GWIN),
        in_specs=[pl.BlockSpec((1, 1, 1, 49, GL), xw_map(j)) for j in range(GWIN)]
        + [
            pl.BlockSpec((1, GWIN, GWIN * 49, 49), lambda b, k, ib, s: (k, 0, 0, 0)),
            pl.BlockSpec((1, GL, GL), lambda b, k, ib, s: (k, 0, 0)),
        ],
        out_specs=[
            pl.BlockSpec((GWIN * 49, GL), lambda b, k, ib, s: (ib, b * KDIR + k)),
            pl.BlockSpec((GWIN * 49, GL), lambda b, k, ib, s: (ib, b * KDIR + k)),
        ],
    )
    return pl.pallas_call(
        _gather_kernel,
        grid_spec=grid_spec,
        out_shape=[
            jax.ShapeDtypeStruct((L_SCAN, DLANE), jnp.float32),
            jax.ShapeDtypeStruct((L_SCAN, DLANE), jnp.float32),
        ],
    )(sel, *([xwp] * GWIN), pstack, mks)


def _scan_chunk_kernel(u_ref, aux_ref, aarr_ref, dtb_ref, ds_ref, e8_ref,
                       y_ref, h_ref, hbuf_ref, da_ref, dbu_ref):
    T = y_ref.shape[0]

    @pl.when(pl.program_id(0) == 0)
    def _():
        h_ref[...] = jnp.zeros_like(h_ref)

    aux = aux_ref[...]
    delta = jax.nn.softplus(aux + dtb_ref[0][None, :])
    du = delta * u_ref[...]
    da_ref[...] = jnp.exp(delta[:, None, :] * aarr_ref[...][None, :, :])
    bc = jnp.stack([aux[:, g * GL + 96:(g + 1) * GL] for g in range(NG)],
                   axis=2)
    dn = (((2,), (0,)), ((), ()))
    bcex = jax.lax.dot_general(bc, e8_ref[...], dn,
                               preferred_element_type=jnp.float32)
    dbu_ref[...] = du[:, None, :] * bcex[:, :16, :]

    def body(t, h):
        h = da_ref[t] * h + dbu_ref[t]
        hbuf_ref[t] = h
        return h

    h = jax.lax.fori_loop(0, T, body, h_ref[...])
    h_ref[...] = h
    y_ref[...] = (jnp.sum(hbuf_ref[...] * bcex[:, 16:, :], axis=1)
                  + u_ref[...] * ds_ref[0][None, :])


def _selective_scan_pallas(u2, aux, aarr, dtb2, ds2, e8):
    L = u2.shape[0]
    T = SCAN_T
    return pl.pallas_call(
        _scan_chunk_kernel,
        grid=(L // T,),
        in_specs=[
            pl.BlockSpec((T, DLANE), lambda i: (i, 0)),
            pl.BlockSpec((T, DLANE), lambda i: (i, 0)),
            pl.BlockSpec((16, DLANE), lambda i: (0, 0)),
            pl.BlockSpec((1, DLANE), lambda i: (0, 0)),
            pl.BlockSpec((1, DLANE), lambda i: (0, 0)),
            pl.BlockSpec((NG, DLANE), lambda i: (0, 0)),
        ],
        out_specs=pl.BlockSpec((T, DLANE), lambda i: (i, 0)),
        out_shape=jax.ShapeDtypeStruct((L, DLANE), jnp.float32),
        scratch_shapes=[
            pltpu.VMEM((16, DLANE), jnp.float32),
            pltpu.VMEM((T, 16, DLANE), jnp.float32),
            pltpu.VMEM((T, 16, DLANE), jnp.float32),
            pltpu.VMEM((T, 16, DLANE), jnp.float32),
        ],
    )(u2, aux, aarr, dtb2, ds2, e8)


def _local_reverse(t, nH, nW, wH, wW, flip=False, column_first=False):
    Bsz, c, L = t.shape
    if flip:
        t = t[..., ::-1]
    if column_first:
        t = jnp.transpose(t.reshape(Bsz, c, nW, nH, wW, wH), (0, 1, 3, 5, 2, 4)).reshape(Bsz, c, L)
    else:
        t = jnp.transpose(t.reshape(Bsz, c, nH, nW, wH, wW), (0, 1, 2, 4, 3, 5)).reshape(Bsz, c, L)
    return t


def _build_constants(x_proj_weight, dt_projs_weight, dt_projs_bias, A_logs, Ds,
                     rw1):
    f32 = jnp.float32
    i96 = jnp.zeros((PERK, GL), f32).at[jnp.arange(PERK), jnp.arange(PERK)].set(1.0)

    p = np.arange(49)
    src = [p, (p % 7) * 7 + p // 7, 48 - p, ((48 - p) % 7) * 7 + (48 - p) // 7]
    pstack = np.zeros((KDIR, GWIN, GWIN * 49, 49), np.float32)
    for k in range(KDIR):
        for j in range(GWIN):
            pstack[k, j, j * 49 + p, src[k]] = 1.0
    pstack = jnp.asarray(pstack)

    xpw = x_proj_weight
    dtw = dt_projs_weight
    mdt = jnp.einsum('krd,ker->kde', xpw[:, :DT_RANK], dtw)
    mks = jnp.zeros((KDIR, GL, GL), f32)
    mks = mks.at[:, :PERK, :PERK].set(mdt)
    mks = mks.at[:, :PERK, PERK:PERK + 16].set(
        jnp.transpose(xpw[:, DT_RANK:DT_RANK + 16], (0, 2, 1)))
    mks = mks.at[:, :PERK, PERK + 16:].set(
        jnp.transpose(xpw[:, DT_RANK + 16:], (0, 2, 1)))

    lane = np.arange(DLANE)
    g = lane // GL
    d = lane % GL
    k_of = g % KDIR
    used = d < PERK
    j_of = np.where(used, k_of * PERK + np.minimum(d, PERK - 1), 0)

    A = -jnp.exp(A_logs)
    usedj = jnp.asarray(used)
    jofj = jnp.asarray(j_of)
    aarr = jnp.where(usedj[None, :], A.T[:, jofj], -1.0)
    dtb = jnp.where(usedj, dt_projs_bias.reshape(-1)[jofj], 0.0)
    dsp = jnp.where(usedj, Ds[jofj], 0.0)
    e8 = (jnp.arange(NG)[:, None] == jnp.asarray(g)[None, :]).astype(f32)

    rw1q = jnp.transpose(rw1.reshape(PERK, PERK, KDIR), (0, 2, 1))
    rw1q = jnp.pad(rw1q, ((0, 0), (0, 0), (0, GL - PERK)))

    return (i96, pstack, mks, aarr.astype(f32),
            dtb.reshape(1, DLANE).astype(f32), dsp.reshape(1, DLANE).astype(f32),
            e8, rw1q)


def kernel(x, x_proj_weight, dt_projs_weight, dt_projs_bias, A_logs, Ds,
           rw1, rb1, rw2, rb2):
    B, C, H, W = x.shape
    n = GRID_N
    L = L_SCAN

    (i96, pstack, mks, aarr, dtbp, dsp, e8, rw1q) = _build_constants(
        x_proj_weight, dt_projs_weight, dt_projs_bias, A_logs, Ds, rw1)

    x5 = x.reshape(B, PERK, KDIR, H, W)
    xwp, pooled = _windowize(x5, i96)

    rw3, self_f, wsel = _route(pooled, rw1q, rb1.reshape(PERK, 1),
                               rw2, rb2.reshape(1, 1))
    if True:
        po = jnp.transpose(x.reshape(B, C, NH, WIN, NW, WIN).mean((3, 5)),
                           (0, 2, 3, 1)).reshape(B, N_WINDOWS, C)
        hx = jax.nn.gelu(po @ rw1.T + rb1, approximate=False)
        lgx = (hx @ rw2.T + rb2)[..., 0]
        orig_rw = jax.nn.softmax(lgx, axis=1)
        routing_weights, sel = jax.lax.top_k(orig_rw, TOP_K)
    else:
        orig_rw = rw3.reshape(B, N_WINDOWS)
        sel = self_f.reshape(B, TOP_K).astype(jnp.int32)
        routing_weights = wsel.reshape(B, TOP_K)

    u2p, aux = _gather(sel, xwp, pstack, mks)
    y2p = _selective_scan_pallas(u2p, aux, aarr, dtbp, dsp, e8)

    out_y = jnp.transpose(
        y2p.reshape(L, B, KDIR, GL)[..., :PERK], (1, 2, 3, 0))

    ys = [
        _local_reverse(out_y[:, 0], n, n, WIN, WIN, flip=False, column_first=False),
        _local_reverse(out_y[:, 1], n, n, WIN, WIN, flip=False, column_first=True),
        _local_reverse(out_y[:, 2], n, n, WIN, WIN, flip=True, column_first=False),
        _local_reverse(out_y[:, 3], n, n, WIN, WIN, flip=True, column_first=True),
    ]
    y = jnp.concatenate(ys, axis=1)
    y = jnp.transpose(y.reshape(B, C, n * n, WIN * WIN), (0, 2, 1, 3)).reshape(B, TOP_K, -1)
    current_state = y * routing_weights[:, :, None]

    windows_flat = jnp.transpose(
        x.reshape(B, C, NH, WIN, NW, WIN), (0, 2, 4, 1, 3, 5)
    ).reshape(B, N_WINDOWS, C * WIN * WIN)
    residual_x = windows_flat * orig_rw[:, :, None]
    residual_x = residual_x.at[jnp.arange(B)[:, None], sel].set(current_state)
    out = jnp.transpose(
        residual_x.reshape(B, NH, NW, C, WIN, WIN), (0, 3, 1, 4, 2, 5)
    ).reshape(B, C, H, W)
    return out


# speedup vs baseline: 6.5823x; 1.0120x over previous
import functools
import math

import jax
import jax.numpy as jnp
import numpy as np
from jax.experimental import pallas as pl
from jax.experimental.pallas import tpu as pltpu

B_, C_, H_, W_ = 2, 384, 224, 224
WIN = 7
D_STATE = 16
DT_RANK = 24
KDIR = 4
PERK = C_ // KDIR

NH = H_ // WIN
NW = W_ // WIN
N_WINDOWS = NH * NW
TOP_K = 256
GRID_N = 16
L_SCAN = TOP_K * WIN * WIN
NG = 2 * KDIR
GL = 128
DLANE = NG * GL

SCAN_T = 64
GWIN = 8


def _windowize_kernel(x_ref, i96_ref, xw_ref, pool_ref):
    xin = x_ref[0, :, 0]
    dn = (((0,), (0,)), ((), ()))
    t = jax.lax.dot_general(xin, i96_ref[...], dn,
                            preferred_element_type=jnp.float32)
    w = t.reshape(8, 7, 32, 7, GL).transpose(0, 2, 1, 3, 4).reshape(256, 49, GL)
    xw_ref[0, 0] = w
    pool_ref[0, 0, 0] = jnp.sum(w, axis=1) * np.float32(1.0 / 49.0)


def _windowize(x5, i96):
    return pl.pallas_call(
        _windowize_kernel,
        grid=(B_, KDIR, 4),
        in_specs=[
            pl.BlockSpec((1, PERK, 1, 56, W_), lambda b, k, hb: (b, 0, k, hb, 0)),
            pl.BlockSpec((PERK, GL), lambda b, k, hb: (0, 0)),
        ],
        out_specs=[
            pl.BlockSpec((1, 1, 256, 49, GL), lambda b, k, hb: (b, k, hb, 0, 0)),
            pl.BlockSpec((1, 1, 1, 256, GL), lambda b, k, hb: (b, k, hb, 0, 0)),
        ],
        out_shape=[
            jax.ShapeDtypeStruct((B_, KDIR, N_WINDOWS, 49, GL), jnp.float32),
            jax.ShapeDtypeStruct((B_, KDIR, 4, 256, GL), jnp.float32),
        ],
    )(x5, i96)


def _gather_kernel(sel_ref, *refs):
    xw = refs[:GWIN]
    pst_ref, mk_ref = refs[GWIN], refs[GWIN + 1]
    u_ref, aux_ref = refs[GWIN + 2], refs[GWIN + 3]
    dn = (((1,), (0,)), ((), ()))
    acc = jnp.zeros((GWIN * 49, GL), jnp.float32)
    for j in range(GWIN):
        acc = acc + jax.lax.dot_general(pst_ref[0, j], xw[j][0, 0, 0], dn,
                                        preferred_element_type=jnp.float32)
    u_ref[...] = acc
    aux_ref[...] = jax.lax.dot_general(acc, mk_ref[0], dn,
                                       preferred_element_type=jnp.float32)


def _gather(sel, xwp, pstack, mks):
    def xw_map(j):
        def f(b, k, ib, sref):
            i = ib * GWIN + j
            i1 = (i % 16) * 16 + i // 16
            i2 = 255 - i
            i3 = (i2 % 16) * 16 + i2 // 16
            idx = jnp.where(k == 0, i,
                            jnp.where(k == 1, i1, jnp.where(k == 2, i2, i3)))
            return (b, k, sref[b, idx], 0, 0)
        return f

    grid_spec = pltpu.PrefetchScalarGridSpec(
        num_scalar_prefetch=1,
        grid=(B_, KDIR, TOP_K // GWIN),
        in_specs=[pl.BlockSpec((1, 1, 1, 49, GL), xw_map(j)) for j in range(GWIN)]
        + [
            pl.BlockSpec((1, GWIN, GWIN * 49, 49), lambda b, k, ib, s: (k, 0, 0, 0)),
            pl.BlockSpec((1, GL, GL), lambda b, k, ib, s: (k, 0, 0)),
        ],
        out_specs=[
            pl.BlockSpec((GWIN * 49, GL), lambda b, k, ib, s: (ib, b * KDIR + k)),
            pl.BlockSpec((GWIN * 49, GL), lambda b, k, ib, s: (ib, b * KDIR + k)),
        ],
    )
    return pl.pallas_call(
        _gather_kernel,
        grid_spec=grid_spec,
        out_shape=[
            jax.ShapeDtypeStruct((L_SCAN, DLANE), jnp.float32),
            jax.ShapeDtypeStruct((L_SCAN, DLANE), jnp.float32),
        ],
    )(sel, *([xwp] * GWIN), pstack, mks)


def _scan_chunk_kernel(u_ref, aux_ref, aarr_ref, dtb_ref, ds_ref, e8_ref,
                       y_ref, h_ref, hbuf_ref, da_ref, dbu_ref):
    T = y_ref.shape[0]

    @pl.when(pl.program_id(0) == 0)
    def _():
        h_ref[...] = jnp.zeros_like(h_ref)

    aux = aux_ref[...]
    delta = jax.nn.softplus(aux + dtb_ref[0][None, :])
    du = delta * u_ref[...]
    da_ref[...] = jnp.exp(delta[:, None, :] * aarr_ref[...][None, :, :])
    bc = jnp.stack([aux[:, g * GL + 96:(g + 1) * GL] for g in range(NG)],
                   axis=2)
    dn = (((2,), (0,)), ((), ()))
    bcex = jax.lax.dot_general(bc, e8_ref[...], dn,
                               preferred_element_type=jnp.float32)
    dbu_ref[...] = du[:, None, :] * bcex[:, :16, :]

    def body(t, h):
        h = da_ref[t] * h + dbu_ref[t]
        hbuf_ref[t] = h
        return h

    h = jax.lax.fori_loop(0, T, body, h_ref[...])
    h_ref[...] = h
    y_ref[...] = (jnp.sum(hbuf_ref[...] * bcex[:, 16:, :], axis=1)
                  + u_ref[...] * ds_ref[0][None, :])


def _selective_scan_pallas(u2, aux, aarr, dtb2, ds2, e8):
    L = u2.shape[0]
    T = SCAN_T
    return pl.pallas_call(
        _scan_chunk_kernel,
        grid=(L // T,),
        in_specs=[
            pl.BlockSpec((T, DLANE), lambda i: (i, 0)),
            pl.BlockSpec((T, DLANE), lambda i: (i, 0)),
            pl.BlockSpec((16, DLANE), lambda i: (0, 0)),
            pl.BlockSpec((1, DLANE), lambda i: (0, 0)),
            pl.BlockSpec((1, DLANE), lambda i: (0, 0)),
            pl.BlockSpec((NG, DLANE), lambda i: (0, 0)),
        ],
        out_specs=pl.BlockSpec((T, DLANE), lambda i: (i, 0)),
        out_shape=jax.ShapeDtypeStruct((L, DLANE), jnp.float32),
        scratch_shapes=[
            pltpu.VMEM((16, DLANE), jnp.float32),
            pltpu.VMEM((T, 16, DLANE), jnp.float32),
            pltpu.VMEM((T, 16, DLANE), jnp.float32),
            pltpu.VMEM((T, 16, DLANE), jnp.float32),
        ],
    )(u2, aux, aarr, dtb2, ds2, e8)


def _local_reverse(t, nH, nW, wH, wW, flip=False, column_first=False):
    Bsz, c, L = t.shape
    if flip:
        t = t[..., ::-1]
    if column_first:
        t = jnp.transpose(t.reshape(Bsz, c, nW, nH, wW, wH), (0, 1, 3, 5, 2, 4)).reshape(Bsz, c, L)
    else:
        t = jnp.transpose(t.reshape(Bsz, c, nH, nW, wH, wW), (0, 1, 2, 4, 3, 5)).reshape(Bsz, c, L)
    return t


def _build_constants(x_proj_weight, dt_projs_weight, dt_projs_bias, A_logs, Ds):
    f32 = jnp.float32
    i96 = jnp.zeros((PERK, GL), f32).at[jnp.arange(PERK), jnp.arange(PERK)].set(1.0)

    p = np.arange(49)
    src = [p, (p % 7) * 7 + p // 7, 48 - p, ((48 - p) % 7) * 7 + (48 - p) // 7]
    pstack = np.zeros((KDIR, GWIN, GWIN * 49, 49), np.float32)
    for k in range(KDIR):
        for j in range(GWIN):
            pstack[k, j, j * 49 + p, src[k]] = 1.0
    pstack = jnp.asarray(pstack)

    xpw = x_proj_weight
    dtw = dt_projs_weight
    mdt = jnp.einsum('krd,ker->kde', xpw[:, :DT_RANK], dtw)
    mks = jnp.zeros((KDIR, GL, GL), f32)
    mks = mks.at[:, :PERK, :PERK].set(mdt)
    mks = mks.at[:, :PERK, PERK:PERK + 16].set(
        jnp.transpose(xpw[:, DT_RANK:DT_RANK + 16], (0, 2, 1)))
    mks = mks.at[:, :PERK, PERK + 16:].set(
        jnp.transpose(xpw[:, DT_RANK + 16:], (0, 2, 1)))

    lane = np.arange(DLANE)
    g = lane // GL
    d = lane % GL
    k_of = g % KDIR
    used = d < PERK
    j_of = np.where(used, k_of * PERK + np.minimum(d, PERK - 1), 0)

    A = -jnp.exp(A_logs)
    usedj = jnp.asarray(used)
    jofj = jnp.asarray(j_of)
    aarr = jnp.where(usedj[None, :], A.T[:, jofj], -1.0)
    dtb = jnp.where(usedj, dt_projs_bias.reshape(-1)[jofj], 0.0)
    dsp = jnp.where(usedj, Ds[jofj], 0.0)
    e8 = (jnp.arange(NG)[:, None] == jnp.asarray(g)[None, :]).astype(f32)

    return (i96, pstack, mks, aarr.astype(f32),
            dtb.reshape(1, DLANE).astype(f32), dsp.reshape(1, DLANE).astype(f32),
            e8)


def kernel(x, x_proj_weight, dt_projs_weight, dt_projs_bias, A_logs, Ds,
           rw1, rb1, rw2, rb2):
    B, C, H, W = x.shape
    n = GRID_N
    L = L_SCAN

    (i96, pstack, mks, aarr, dtbp, dsp, e8) = _build_constants(
        x_proj_weight, dt_projs_weight, dt_projs_bias, A_logs, Ds)

    x5 = x.reshape(B, PERK, KDIR, H, W)
    xwp, pooled = _windowize(x5, i96)

    po = jnp.transpose(pooled[..., :PERK], (0, 2, 3, 4, 1)).reshape(B, N_WINDOWS, C)
    hx = jax.nn.gelu(po @ rw1.T + rb1, approximate=False)
    lgx = (hx @ rw2.T + rb2)[..., 0]
    orig_rw = jax.nn.softmax(lgx, axis=1)
    routing_weights, sel = jax.lax.top_k(orig_rw, TOP_K)

    u2p, aux = _gather(sel, xwp, pstack, mks)
    y2p = _selective_scan_pallas(u2p, aux, aarr, dtbp, dsp, e8)

    out_y = jnp.transpose(
        y2p.reshape(L, B, KDIR, GL)[..., :PERK], (1, 2, 3, 0))

    ys = [
        _local_reverse(out_y[:, 0], n, n, WIN, WIN, flip=False, column_first=False),
        _local_reverse(out_y[:, 1], n, n, WIN, WIN, flip=False, column_first=True),
        _local_reverse(out_y[:, 2], n, n, WIN, WIN, flip=True, column_first=False),
        _local_reverse(out_y[:, 3], n, n, WIN, WIN, flip=True, column_first=True),
    ]
    y = jnp.concatenate(ys, axis=1)
    y = jnp.transpose(y.reshape(B, C, n * n, WIN * WIN), (0, 2, 1, 3)).reshape(B, TOP_K, -1)
    current_state = y * routing_weights[:, :, None]

    windows_flat = jnp.transpose(
        x.reshape(B, C, NH, WIN, NW, WIN), (0, 2, 4, 1, 3, 5)
    ).reshape(B, N_WINDOWS, C * WIN * WIN)
    residual_x = windows_flat * orig_rw[:, :, None]
    residual_x = residual_x.at[jnp.arange(B)[:, None], sel].set(current_state)
    out = jnp.transpose(
        residual_x.reshape(B, NH, NW, C, WIN, WIN), (0, 3, 1, 4, 2, 5)
    ).reshape(B, C, H, W)
    return out
